# Initial kernel scaffold; baseline (speedup 1.0000x reference)
import jax
import jax.numpy as jnp
from jax.experimental import pallas as pl


def kernel(W, queries, query_lengths, num_cands, x_type_bow, x_type_bow_len, x_path_bow, x_path_bow_len, x_ctx_ent, x_ctx_ent_len, x_ctx_ent_num):
    raise NotImplementedError("write your pallas kernel here")


# speedup vs baseline: 13.2738x; 13.2738x over previous
import functools

import jax
import jax.numpy as jnp
import numpy as np
from jax import lax
from jax.experimental import pallas as pl
from jax.experimental.pallas import tpu as pltpu
from jax.experimental.pallas import tpu_sc as plsc

_VOCAB = 100000
_D = 64
_B = 1024
_C = 20
_NCTX = 8
_LT = 3
_LP = 8
_LC = 5
_LQ = 50
_INF = 1e20

_NZPAD = 1024
_VPAD = _VOCAB + _NZPAD
_ROWW = 128

_TPC = _LT + _LP + _NCTX * _LC
_NTOK = _LQ + _C * _TPC
_CHUNK = 128
_NCHUNK = 9
_NPAD = _NCHUNK * _CHUNK
_HCH = 5
_HOFF = 4
_NLEN = 224

_OFF_QL = 0
_OFF_NC = 1
_OFF_TL = 2
_OFF_PL = 22
_OFF_NUM = 42
_OFF_CL = 62
_OFF_PAD = 222

_NW = 32
_BPW = _B // _NW


def _build_consts():
    pos = np.ones((_NPAD,), np.int32)
    off = np.full((_NPAD,), _OFF_PAD, np.int32)
    pos[0:_LQ] = np.arange(_LQ)
    off[0:_LQ] = _OFF_QL
    for c in range(_C):
        b = _LQ + _TPC * c
        pos[b:b + _LT] = np.arange(_LT)
        off[b:b + _LT] = _OFF_TL + c
        pos[b + _LT:b + _LT + _LP] = np.arange(_LP)
        off[b + _LT:b + _LT + _LP] = _OFF_PL + c
        for n in range(_NCTX):
            bb = b + _LT + _LP + _LC * n
            pos[bb:bb + _LC] = np.arange(_LC)
            off[bb:bb + _LC] = _OFF_CL + _NCTX * c + n
    spread = (_VOCAB + (np.arange(_NPAD, dtype=np.int64) * 89) % _NZPAD
              ).astype(np.int32)
    shp = (_NCHUNK, _CHUNK)
    return pos.reshape(shp), off.reshape(shp), spread.reshape(shp)


_POS_NP, _OFF_NP, _SPR_NP = _build_consts()
_GBASE_NP = np.array(
    [_OFF_TL, _OFF_PL, _OFF_NUM] + [_OFF_CL + n for n in range(_NCTX)]
    + [_OFF_PAD] * 5, np.int32)
_GMULT_NP = np.array([1, 1, 1] + [_NCTX] * _NCTX + [0] * 5, np.int32)
_AUX_NP = np.stack([np.arange(16, dtype=np.int32), _GBASE_NP, _GMULT_NP,
                    np.zeros(16, np.int32)])


def _bownet_sc_body(w_hbm, idx_hbm, lens_hbm, pos_hbm, off_hbm, spr_hbm,
                    aux_hbm, out_hbm, pos_v, off_v, spr_v, idx_v, lens_v,
                    rows_v, score_v, aux_v, sem):
    wid = lax.axis_index("s") * 2 + lax.axis_index("c")
    pltpu.sync_copy(pos_hbm, pos_v)
    pltpu.sync_copy(off_hbm, off_v)
    pltpu.sync_copy(spr_hbm, spr_v)
    pltpu.sync_copy(aux_hbm, aux_v)

    @pl.loop(0, _BPW)
    def _batch(bl):
        b = wid * _BPW + bl
        pltpu.sync_copy(idx_hbm.at[b], idx_v)
        pltpu.sync_copy(lens_hbm.at[b], lens_v)

        @pl.loop(0, _NCHUNK)
        def _mask(j):
            for k in range(_CHUNK // 16):
                sl = pl.ds(k * 16, 16)
                raw = idx_v[j, sl]
                ln = plsc.load_gather(lens_v, [off_v[j, sl]])
                idx_v[j, sl] = jnp.where(pos_v[j, sl] < ln, raw, spr_v[j, sl])

        lv0 = lens_v[pl.ds(0, 16)]
        rlv0 = 1.0 / lv0.astype(jnp.float32)
        rql = rlv0[_OFF_QL]
        nc = lv0[_OFF_NC]
        ntok = _LQ + _TPC * nc
        nch = (ntok + _CHUNK - 1) // _CHUNK
        nch0 = jnp.minimum(nch, _HCH)

        def _fire(j, dst_base):
            return pltpu.async_copy(
                w_hbm.at[idx_v.at[j]],
                rows_v.at[pl.ds((j - dst_base) * _CHUNK, _CHUNK)], sem)

        @pl.loop(0, nch0)
        def _fire0(j):
            _fire(j, 0)

        @pl.loop(0, nch0)
        def _drain0(j):
            pltpu.make_async_copy(
                w_hbm.at[idx_v.at[0]], rows_v.at[pl.ds(0, _CHUNK)], sem
            ).wait()

        def _qacc(t, acc):
            return tuple(acc[k] + rows_v[t, pl.ds(k * 16, 16)]
                         for k in range(4))
        z = jnp.zeros((16,), jnp.float32)
        qs = lax.fori_loop(0, _LQ, _qacc, (z, z, z, z))
        q = tuple(a * rql for a in qs)

        iota = aux_v[0, :]
        gbase = aux_v[1, :]
        gmult = aux_v[2, :]

        def _cand(h):
            def body(c, carry):
                s0, s1 = carry
                g = plsc.load_gather(lens_v, [gbase + c * gmult])
                r = 1.0 / g.astype(jnp.float32)
                rt = r[0]
                rp = r[1]
                rn = r[2]
                num = g[2]
                wv = jnp.where((iota >= 3) & (iota < 3 + num), rn * r, 0.0)
                bt = _LQ + _TPC * c - h * _HOFF * _CHUNK
                bp = bt + _LT
                bc = bp + _LP
                key = []
                for k in range(4):
                    sl = pl.ds(k * 16, 16)
                    ts = (rows_v[bt, sl] + rows_v[bt + 1, sl]
                          + rows_v[bt + 2, sl])
                    ps = rows_v[bp, sl]
                    for j in range(1, _LP):
                        ps = ps + rows_v[bp + j, sl]
                    key.append(ts * rt + ps * rp)
                for n in range(_NCTX):
                    w = wv[3 + n]
                    base = bc + _LC * n
                    for k in range(4):
                        sl = pl.ds(k * 16, 16)
                        s = rows_v[base, sl]
                        for j in range(1, _LC):
                            s = s + rows_v[base + j, sl]
                        key[k] = key[k] + w * s
                v = (key[0] * q[0] + key[1] * q[1] + key[2] * q[2]
                     + key[3] * q[3])
                sc = jnp.sum(v)
                s0 = jnp.where(iota == c, sc, s0)
                s1 = jnp.where(iota == c - 16, sc, s1)
                return s0, s1
            return body

        neg = jnp.full((16,), -_INF, jnp.float32)
        s0, s1 = lax.fori_loop(0, jnp.minimum(nc, _C // 2), _cand(0),
                               (neg, neg))
        score_v[bl, pl.ds(0, 16)] = s0
        score_v[bl, pl.ds(16, 16)] = s1

        @pl.when(nc > _C // 2)
        def _pass1():
            @pl.loop(_HOFF, nch)
            def _fire1(j):
                _fire(j, _HOFF)

            @pl.loop(_HOFF, nch)
            def _drain1(j):
                pltpu.make_async_copy(
                    w_hbm.at[idx_v.at[0]], rows_v.at[pl.ds(0, _CHUNK)], sem
                ).wait()

            t0 = score_v[bl, pl.ds(0, 16)]
            t1 = score_v[bl, pl.ds(16, 16)]
            u0, u1 = lax.fori_loop(_C // 2, nc, _cand(1), (t0, t1))
            score_v[bl, pl.ds(0, 16)] = u0
            score_v[bl, pl.ds(16, 16)] = u1

    pltpu.sync_copy(score_v, out_hbm.at[pl.ds(wid * _BPW, _BPW)])


@functools.lru_cache(maxsize=None)
def _get_sc_call():
    mesh = plsc.VectorSubcoreMesh(
        core_axis_name="c", subcore_axis_name="s",
        num_cores=2, num_subcores=16)
    return pl.kernel(
        _bownet_sc_body,
        out_type=jax.ShapeDtypeStruct((_B, 32), jnp.float32),
        mesh=mesh,
        compiler_params=pltpu.CompilerParams(needs_layout_passes=False),
        scratch_types=[
            pltpu.VMEM((_NCHUNK, _CHUNK), jnp.int32),
            pltpu.VMEM((_NCHUNK, _CHUNK), jnp.int32),
            pltpu.VMEM((_NCHUNK, _CHUNK), jnp.int32),
            pltpu.VMEM((_NCHUNK, _CHUNK), jnp.int32),
            pltpu.VMEM((_NLEN,), jnp.int32),
            pltpu.VMEM((_HCH * _CHUNK, _ROWW), jnp.float32),
            pltpu.VMEM((_BPW, 32), jnp.float32),
            pltpu.VMEM((4, 16), jnp.int32),
            pltpu.SemaphoreType.DMA,
        ],
    )


def _candidate_major(x_type_bow, x_path_bow, x_ctx_ent):
    return jnp.concatenate([
        x_type_bow,
        x_path_bow,
        x_ctx_ent.reshape(_B, _C, _NCTX * _LC),
    ], axis=2)


def kernel(W, queries, query_lengths, num_cands, x_type_bow, x_type_bow_len,
           x_path_bow, x_path_bow_len, x_ctx_ent, x_ctx_ent_len,
           x_ctx_ent_num):
    i32 = jnp.int32
    Wp = jnp.pad(W, ((0, _NZPAD), (0, _ROWW - _D)))
    cand = _candidate_major(x_type_bow, x_path_bow, x_ctx_ent)
    idx = jnp.concatenate([
        queries,
        cand.reshape(_B, _C * _TPC),
        jnp.zeros((_B, _NPAD - _NTOK), i32),
    ], axis=1).reshape(_B, _NCHUNK, _CHUNK)
    lens = jnp.concatenate([
        query_lengths[:, None],
        num_cands[:, None],
        x_type_bow_len,
        x_path_bow_len,
        x_ctx_ent_num,
        x_ctx_ent_len.reshape(_B, _C * _NCTX),
        jnp.ones((_B, 2), i32),
    ], axis=1)
    out = _get_sc_call()(Wp, idx, lens, jnp.asarray(_POS_NP),
                         jnp.asarray(_OFF_NP), jnp.asarray(_SPR_NP),
                         jnp.asarray(_AUX_NP))
    return out[:, :_C]
